# baseline (speedup 1.0000x reference)
import functools

import jax
import jax.numpy as jnp
from jax import lax
from jax.experimental import pallas as pl
from jax.experimental.pallas import tpu as pltpu
from jax.experimental.pallas import tpu_sc as plsc

NUM_EMB = 1_000_000
D = 64
S = 16384
W = 50
NC, NS = 2, 16
NW = NC * NS
SB = S // 128
SB_PER_W = SB // NW
QH = 2
CH = QH * 128
N_CHUNKS = W * (SB_PER_W // QH)


def _gather(table, idx3):
    mesh = plsc.VectorSubcoreMesh(core_axis_name="c", subcore_axis_name="s")

    @functools.partial(
        pl.kernel,
        out_type=jax.ShapeDtypeStruct((W, 8, SB, 8, 128), jnp.float32),
        mesh=mesh,
        scratch_types=[
            pltpu.VMEM((2, QH, 128), jnp.int32),
            pltpu.VMEM((2, CH, D), jnp.float32),
            pltpu.VMEM((2, 8, QH, 8, 128), jnp.float32),
            pltpu.SemaphoreType.DMA((2,)),
            pltpu.SemaphoreType.DMA((2,)),
        ],
        compiler_params=pltpu.CompilerParams(
            use_tc_tiling_on_sc=False, needs_layout_passes=False),
    )
    def k(idx_hbm, table_hbm, out_hbm, idx_v, rows_v, trans_v, sem_g, sem_w):
        wid = lax.axis_index("s") * NC + lax.axis_index("c")
        s_base = SB_PER_W * wid

        def fire_gather(c, b):
            w = c // QH
            s0 = s_base + QH * (c % QH)
            pltpu.sync_copy(idx_hbm.at[w, pl.ds(s0, QH)], idx_v.at[b])
            for q in range(QH):
                pltpu.async_copy(
                    table_hbm.at[idx_v.at[b, q]],
                    rows_v.at[b, pl.ds(q * 128, 128)], sem_g.at[b])

        def wait_gather(b):
            pltpu.make_async_copy(
                table_hbm.at[pl.ds(0, CH)], rows_v.at[b], sem_g.at[b]).wait()

        def transpose(b):
            for q in range(QH):
                for g in range(8):
                    row = lax.iota(jnp.int32, 16) + (q * 128 + g * 16)

                    @pl.loop(0, 8)
                    def dblk(dh):
                        for dl in range(8):
                            col = jnp.zeros((16,), jnp.int32) + (dh * 8 + dl)
                            v = plsc.load_gather(rows_v.at[b], [row, col])
                            trans_v[b, dh, q, dl, pl.ds(g * 16, 16)] = v

        def fire_wb(c, b):
            w = c // QH
            s0 = s_base + QH * (c % QH)
            for dh in range(8):
                pltpu.async_copy(
                    trans_v.at[b, dh],
                    out_hbm.at[w, dh, pl.ds(s0, QH)], sem_w.at[b])

        def wait_wb(b):
            pltpu.make_async_copy(
                trans_v.at[b], out_hbm.at[0, pl.ds(0, 8), pl.ds(0, QH)],
                sem_w.at[b]).wait()

        fire_gather(0, 0)

        @pl.loop(0, N_CHUNKS, step=2)
        def steady(c0):
            for j in range(2):
                c = c0 + j
                b = j
                b1 = 1 - j
                @pl.when(c + 1 < N_CHUNKS)
                def _():
                    @pl.when(c >= 1)
                    def _():
                        wait_wb(b1)
                    fire_gather(c + 1, b1)

                wait_gather(b)
                transpose(b)
                fire_wb(c, b)

        for b in range(2):
            wait_wb(b)

    return k(idx3, table)


@jax.jit
def kernel(input_vector, table):
    idx3 = input_vector.T.reshape(W, SB, 128)
    out5 = _gather(table, idx3)
    return out5.transpose(2, 4, 0, 1, 3).reshape(S, W, D)


# speedup vs baseline: 1.0329x; 1.0329x over previous
import functools

import jax
import jax.numpy as jnp
from jax import lax
from jax.experimental import pallas as pl
from jax.experimental.pallas import tpu as pltpu
from jax.experimental.pallas import tpu_sc as plsc

NUM_EMB = 1_000_000
D = 64
S = 16384
W = 50
NC, NS = 2, 16
NW = NC * NS
SB = S // 128
SB_PER_W = SB // NW
QH = 2
CH = QH * 128
N_CHUNKS = W * (SB_PER_W // QH)


def _gather(table, idx3):
    mesh = plsc.VectorSubcoreMesh(core_axis_name="c", subcore_axis_name="s")

    @functools.partial(
        pl.kernel,
        out_type=jax.ShapeDtypeStruct((W, 8, SB, 8, 128), jnp.float32),
        mesh=mesh,
        scratch_types=[
            pltpu.VMEM((W, SB_PER_W, 128), jnp.int32),
            pltpu.VMEM((2, CH, D), jnp.float32),
            pltpu.VMEM((2, 8, QH, 8, 128), jnp.float32),
            pltpu.SemaphoreType.DMA((2,)),
            pltpu.SemaphoreType.DMA((2,)),
        ],
        compiler_params=pltpu.CompilerParams(
            use_tc_tiling_on_sc=False, needs_layout_passes=False),
    )
    def k(idx_hbm, table_hbm, out_hbm, idx_v, rows_v, trans_v, sem_g, sem_w):
        wid = lax.axis_index("s") * NC + lax.axis_index("c")
        s_base = SB_PER_W * wid

        pltpu.sync_copy(idx_hbm.at[:, pl.ds(s_base, SB_PER_W)], idx_v)

        def fire_gather(c, b):
            w = c // QH
            h = c % QH
            for q in range(QH):
                pltpu.async_copy(
                    table_hbm.at[idx_v.at[w, h * QH + q]],
                    rows_v.at[b, pl.ds(q * 128, 128)], sem_g.at[b])

        def wait_gather(b):
            pltpu.make_async_copy(
                table_hbm.at[pl.ds(0, CH)], rows_v.at[b], sem_g.at[b]).wait()

        def transpose(b):
            for q in range(QH):
                @pl.loop(0, 8)
                def gloop(g):
                    row = lax.iota(jnp.int32, 16) + (g * 16 + q * 128)
                    for dh in range(8):
                        for dl in range(8):
                            col = jnp.full((16,), dh * 8 + dl, jnp.int32)
                            v = plsc.load_gather(rows_v.at[b], [row, col])
                            trans_v[b, dh, q, dl, pl.ds(g * 16, 16)] = v

        def fire_wb(c, b):
            w = c // QH
            s0 = s_base + QH * (c % QH)
            for dh in range(8):
                pltpu.async_copy(
                    trans_v.at[b, dh],
                    out_hbm.at[w, dh, pl.ds(s0, QH)], sem_w.at[b])

        def wait_wb(b):
            pltpu.make_async_copy(
                trans_v.at[b], out_hbm.at[0, pl.ds(0, 8), pl.ds(0, QH)],
                sem_w.at[b]).wait()

        fire_gather(0, 0)

        @pl.loop(0, N_CHUNKS, step=2)
        def steady(c0):
            for j in range(2):
                c = c0 + j
                b = j
                b1 = 1 - j
                @pl.when(c + 1 < N_CHUNKS)
                def _():
                    @pl.when(c >= 1)
                    def _():
                        wait_wb(b1)
                    fire_gather(c + 1, b1)

                wait_gather(b)
                transpose(b)
                fire_wb(c, b)

        for b in range(2):
            wait_wb(b)

    return k(idx3, table)


@jax.jit
def kernel(input_vector, table):
    idx3 = input_vector.T.reshape(W, SB, 128)
    out5 = _gather(table, idx3)
    return out5.transpose(2, 4, 0, 1, 3).reshape(S, W, D)


# speedup vs baseline: 1.4345x; 1.3888x over previous
import functools

import jax
import jax.numpy as jnp
from jax import lax
from jax.experimental import pallas as pl
from jax.experimental.pallas import tpu as pltpu
from jax.experimental.pallas import tpu_sc as plsc

NUM_EMB = 1_000_000
D = 64
S = 16384
W = 50
NC, NS = 2, 16
NW = NC * NS
SB = S // 128
SB_PER_W = SB // NW
QH = 2
CH = QH * 128
N_CHUNKS = W * (SB_PER_W // QH)


def _gather(table, idx3):
    mesh = plsc.VectorSubcoreMesh(core_axis_name="c", subcore_axis_name="s")

    @functools.partial(
        pl.kernel,
        out_type=jax.ShapeDtypeStruct((W, 8, SB, 8, 128), jnp.float32),
        mesh=mesh,
        scratch_types=[
            pltpu.VMEM((W, SB_PER_W, 128), jnp.int32),
            pltpu.VMEM((2, CH, D), jnp.float32),
            pltpu.VMEM((2, 8, QH, 8, 128), jnp.float32),
            pltpu.SemaphoreType.DMA((2,)),
            pltpu.SemaphoreType.DMA((2,)),
        ],
        compiler_params=pltpu.CompilerParams(
            use_tc_tiling_on_sc=False, needs_layout_passes=False),
    )
    def k(idx_hbm, table_hbm, out_hbm, idx_v, rows_v, trans_v, sem_g, sem_w):
        wid = lax.axis_index("s") * NC + lax.axis_index("c")
        s_base = SB_PER_W * wid

        pltpu.sync_copy(idx_hbm.at[:, pl.ds(s_base, SB_PER_W)], idx_v)

        def fire_gather(c, b):
            w = c // QH
            h = c % QH
            for q in range(QH):
                pltpu.async_copy(
                    table_hbm.at[idx_v.at[w, h * QH + q]],
                    rows_v.at[b, pl.ds(q * 128, 128)], sem_g.at[b])

        def wait_gather(b):
            pltpu.make_async_copy(
                table_hbm.at[pl.ds(0, CH)], rows_v.at[b], sem_g.at[b]).wait()

        def transpose(b):
            for q in range(QH):
                @plsc.parallel_loop(0, 8, unroll=2)
                def gloop(g):
                    row = lax.iota(jnp.int32, 16) + (g * 16 + q * 128)
                    for dh in range(8):
                        vs = [
                            plsc.load_gather(
                                rows_v.at[b],
                                [row, jnp.full((16,), dh * 8 + dl, jnp.int32)])
                            for dl in range(8)
                        ]
                        for dl in range(8):
                            trans_v[b, dh, q, dl, pl.ds(g * 16, 16)] = vs[dl]

        def fire_wb(c, b):
            w = c // QH
            s0 = s_base + QH * (c % QH)
            for dh in range(8):
                pltpu.async_copy(
                    trans_v.at[b, dh],
                    out_hbm.at[w, dh, pl.ds(s0, QH)], sem_w.at[b])

        def wait_wb(b):
            pltpu.make_async_copy(
                trans_v.at[b], out_hbm.at[0, pl.ds(0, 8), pl.ds(0, QH)],
                sem_w.at[b]).wait()

        fire_gather(0, 0)

        @pl.loop(0, N_CHUNKS, step=2)
        def steady(c0):
            for j in range(2):
                c = c0 + j
                b = j
                b1 = 1 - j
                @pl.when(c + 1 < N_CHUNKS)
                def _():
                    @pl.when(c >= 1)
                    def _():
                        wait_wb(b1)
                    fire_gather(c + 1, b1)

                wait_gather(b)
                transpose(b)
                fire_wb(c, b)

        for b in range(2):
            wait_wb(b)

    return k(idx3, table)


@jax.jit
def kernel(input_vector, table):
    idx3 = input_vector.T.reshape(W, SB, 128)
    out5 = _gather(table, idx3)
    return out5.transpose(2, 4, 0, 1, 3).reshape(S, W, D)


# speedup vs baseline: 2.4242x; 1.6900x over previous
import functools

import jax
import jax.numpy as jnp
from jax import lax
from jax.experimental import pallas as pl
from jax.experimental.pallas import tpu as pltpu
from jax.experimental.pallas import tpu_sc as plsc

NUM_EMB = 1_000_000
D = 64
S = 16384
W = 50
NC, NS = 2, 16
NW = NC * NS
SB = S // 128
SB_PER_W = SB // NW
QH = 2
CH = QH * 128
N_CHUNKS = W * (SB_PER_W // QH)


def _gather(table, idx3):
    mesh = plsc.VectorSubcoreMesh(core_axis_name="c", subcore_axis_name="s")

    @functools.partial(
        pl.kernel,
        out_type=jax.ShapeDtypeStruct((W, 8, SB, 8, 128), jnp.float32),
        mesh=mesh,
        scratch_types=[
            pltpu.VMEM((W, SB_PER_W, 128), jnp.int32),
            pltpu.VMEM((2, CH, D), jnp.float32),
            pltpu.VMEM((D * 257,), jnp.float32),
            pltpu.VMEM((2, 8, QH, 8, 128), jnp.float32),
            pltpu.SemaphoreType.DMA((2,)),
            pltpu.SemaphoreType.DMA((2,)),
        ],
        compiler_params=pltpu.CompilerParams(
            use_tc_tiling_on_sc=False, needs_layout_passes=False),
    )
    def k(idx_hbm, table_hbm, out_hbm, idx_v, rows_v, skew_v, trans_v,
          sem_g, sem_w):
        wid = lax.axis_index("s") * NC + lax.axis_index("c")
        s_base = SB_PER_W * wid

        pltpu.sync_copy(idx_hbm.at[:, pl.ds(s_base, SB_PER_W)], idx_v)

        def fire_gather(c, b):
            w = c // QH
            h = c % QH
            for q in range(QH):
                pltpu.async_copy(
                    table_hbm.at[idx_v.at[w, h * QH + q]],
                    rows_v.at[b, pl.ds(q * 128, 128)], sem_g.at[b])

        def wait_gather(b):
            pltpu.make_async_copy(
                table_hbm.at[pl.ds(0, CH)], rows_v.at[b], sem_g.at[b]).wait()

        def transpose(b):
            pats = [(lax.iota(jnp.int32, 16) + d0) * 257 for d0 in range(0, D, 16)]

            @plsc.parallel_loop(0, CH, unroll=4)
            def tloop(t):
                for i in range(D // 16):
                    v = rows_v[b, t, pl.ds(i * 16, 16)]
                    plsc.store_scatter(skew_v, [pats[i] + t], v)

            for q in range(QH):
                @plsc.parallel_loop(0, 8, unroll=2)
                def gloop(g):
                    base = q * 128 + g * 16
                    for dh in range(8):
                        vs = [
                            skew_v[pl.ds((dh * 8 + dl) * 257 + base, 16)]
                            for dl in range(8)
                        ]
                        for dl in range(8):
                            trans_v[b, dh, q, dl, pl.ds(g * 16, 16)] = vs[dl]

        def fire_wb(c, b):
            w = c // QH
            s0 = s_base + QH * (c % QH)
            for dh in range(8):
                pltpu.async_copy(
                    trans_v.at[b, dh],
                    out_hbm.at[w, dh, pl.ds(s0, QH)], sem_w.at[b])

        def wait_wb(b):
            pltpu.make_async_copy(
                trans_v.at[b], out_hbm.at[0, pl.ds(0, 8), pl.ds(0, QH)],
                sem_w.at[b]).wait()

        fire_gather(0, 0)

        @pl.loop(0, N_CHUNKS, step=2)
        def steady(c0):
            for j in range(2):
                c = c0 + j
                b = j
                b1 = 1 - j
                @pl.when(c + 1 < N_CHUNKS)
                def _():
                    @pl.when(c >= 1)
                    def _():
                        wait_wb(b1)
                    fire_gather(c + 1, b1)

                wait_gather(b)
                transpose(b)
                fire_wb(c, b)

        for b in range(2):
            wait_wb(b)

    return k(idx3, table)


@jax.jit
def kernel(input_vector, table):
    idx3 = input_vector.T.reshape(W, SB, 128)
    out5 = _gather(table, idx3)
    return out5.transpose(2, 4, 0, 1, 3).reshape(S, W, D)


# speedup vs baseline: 3.9217x; 1.6177x over previous
import functools

import jax
import jax.numpy as jnp
from jax import lax
from jax.experimental import pallas as pl
from jax.experimental.pallas import tpu as pltpu
from jax.experimental.pallas import tpu_sc as plsc

NUM_EMB = 1_000_000
D = 64
S = 16384
W = 50
NC, NS = 2, 16
NW = NC * NS
SB = S // 128
SB_PER_W = SB // NW
QH = 2
CH = QH * 128
N_CHUNKS = W * (SB_PER_W // QH)

E_FULL = (NUM_EMB // 128) * 128
NCOL = E_FULL // 128
COLS_PER_W = NCOL // NW
NCOL_EXTRA = NCOL - COLS_PER_W * NW
E_TAIL = NUM_EMB - E_FULL


def _relayout(tableT, tail):
    mesh = plsc.VectorSubcoreMesh(core_axis_name="c", subcore_axis_name="s")

    @functools.partial(
        pl.kernel,
        out_type=jax.ShapeDtypeStruct((NUM_EMB * D,), jnp.float32),
        mesh=mesh,
        scratch_types=[
            pltpu.VMEM((2, D, 128), jnp.float32),
            pltpu.VMEM((2, 128 * D), jnp.float32),
            pltpu.VMEM((128 * 65,), jnp.float32),
            pltpu.SemaphoreType.DMA((2,)),
            pltpu.SemaphoreType.DMA((2,)),
        ],
        compiler_params=pltpu.CompilerParams(
            use_tc_tiling_on_sc=True, needs_layout_passes=False),
    )
    def ka(tT_hbm, tail_hbm, out_hbm, blk, ob, skew, sem_i, sem_o):
        wid = lax.axis_index("s") * NC + lax.axis_index("c")
        start = wid * COLS_PER_W

        def fire_in(c, b):
            pltpu.async_copy(
                tT_hbm.at[:, pl.ds((start + c) * 128, 128)], blk.at[b],
                sem_i.at[b])

        def wait_in(b):
            pltpu.make_async_copy(
                tT_hbm.at[:, pl.ds(0, 128)], blk.at[b], sem_i.at[b]).wait()

        pats = [(lax.iota(jnp.int32, 16) + g * 16) * 65 for g in range(8)]

        def trans(b):
            @plsc.parallel_loop(0, D, unroll=2)
            def s1(d):
                for g in range(8):
                    v = blk[b, d, pl.ds(g * 16, 16)]
                    plsc.store_scatter(skew, [pats[g] + d], v)

            @plsc.parallel_loop(0, 128, unroll=2)
            def s2(e):
                for i in range(D // 16):
                    ob[b, pl.ds(e * D + i * 16, 16)] = (
                        skew[pl.ds(e * 65 + i * 16, 16)])

        def fire_out(c, b):
            pltpu.async_copy(
                ob.at[b], out_hbm.at[pl.ds((start + c) * 128 * D, 128 * D)],
                sem_o.at[b])

        def wait_out(b):
            pltpu.make_async_copy(
                ob.at[b], out_hbm.at[pl.ds(0, 128 * D)], sem_o.at[b]).wait()

        fire_in(0, 0)

        @pl.loop(0, COLS_PER_W, step=2)
        def steady(c0):
            for j in range(2):
                c = c0 + j
                b = j
                b1 = 1 - j

                @pl.when(c + 1 < COLS_PER_W)
                def _():
                    @pl.when(c >= 1)
                    def _():
                        wait_out(b1)
                    fire_in(c + 1, b1)

                wait_in(b)
                trans(b)
                fire_out(c, b)

        for b in range(2):
            wait_out(b)

        @pl.when(wid < NCOL_EXTRA)
        def _():
            c = NCOL - NCOL_EXTRA - start + wid
            fire_in(c, 0)
            wait_in(0)
            trans(0)
            fire_out(c, 0)
            wait_out(0)

        @pl.when(wid == 0)
        def _():
            pltpu.sync_copy(tail_hbm, ob.at[0, pl.ds(0, E_TAIL * D)])
            pltpu.sync_copy(ob.at[0, pl.ds(0, E_TAIL * D)],
                            out_hbm.at[pl.ds(E_FULL * D, E_TAIL * D)])

    return ka(tableT, tail)


def _gather(table, idx3):
    mesh = plsc.VectorSubcoreMesh(core_axis_name="c", subcore_axis_name="s")

    @functools.partial(
        pl.kernel,
        out_type=jax.ShapeDtypeStruct((W, 8, SB, 8, 128), jnp.float32),
        mesh=mesh,
        scratch_types=[
            pltpu.VMEM((W, SB_PER_W, 128), jnp.int32),
            pltpu.VMEM((2, CH, D), jnp.float32),
            pltpu.VMEM((D * 257,), jnp.float32),
            pltpu.VMEM((2, 8, QH, 8, 128), jnp.float32),
            pltpu.SemaphoreType.DMA((2,)),
            pltpu.SemaphoreType.DMA((2,)),
        ],
        compiler_params=pltpu.CompilerParams(
            use_tc_tiling_on_sc=False, needs_layout_passes=False),
    )
    def k(idx_hbm, table_hbm, out_hbm, idx_v, rows_v, skew_v, trans_v,
          sem_g, sem_w):
        wid = lax.axis_index("s") * NC + lax.axis_index("c")
        s_base = SB_PER_W * wid

        pltpu.sync_copy(idx_hbm.at[:, pl.ds(s_base, SB_PER_W)], idx_v)

        def fire_gather(c, b):
            w = c // QH
            h = c % QH
            for q in range(QH):
                pltpu.async_copy(
                    table_hbm.at[idx_v.at[w, h * QH + q]],
                    rows_v.at[b, pl.ds(q * 128, 128)], sem_g.at[b])

        def wait_gather(b):
            pltpu.make_async_copy(
                table_hbm.at[pl.ds(0, CH)], rows_v.at[b], sem_g.at[b]).wait()

        def transpose(b):
            pats = [(lax.iota(jnp.int32, 16) + d0) * 257 for d0 in range(0, D, 16)]

            @plsc.parallel_loop(0, CH, unroll=4)
            def tloop(t):
                for i in range(D // 16):
                    v = rows_v[b, t, pl.ds(i * 16, 16)]
                    plsc.store_scatter(skew_v, [pats[i] + t], v)

            for q in range(QH):
                @plsc.parallel_loop(0, 8, unroll=2)
                def gloop(g):
                    base = q * 128 + g * 16
                    for dh in range(8):
                        vs = [
                            skew_v[pl.ds((dh * 8 + dl) * 257 + base, 16)]
                            for dl in range(8)
                        ]
                        for dl in range(8):
                            trans_v[b, dh, q, dl, pl.ds(g * 16, 16)] = vs[dl]

        def fire_wb(c, b):
            w = c // QH
            s0 = s_base + QH * (c % QH)
            for dh in range(8):
                pltpu.async_copy(
                    trans_v.at[b, dh],
                    out_hbm.at[w, dh, pl.ds(s0, QH)], sem_w.at[b])

        def wait_wb(b):
            pltpu.make_async_copy(
                trans_v.at[b], out_hbm.at[0, pl.ds(0, 8), pl.ds(0, QH)],
                sem_w.at[b]).wait()

        fire_gather(0, 0)

        @pl.loop(0, N_CHUNKS, step=2)
        def steady(c0):
            for j in range(2):
                c = c0 + j
                b = j
                b1 = 1 - j
                @pl.when(c + 1 < N_CHUNKS)
                def _():
                    @pl.when(c >= 1)
                    def _():
                        wait_wb(b1)
                    fire_gather(c + 1, b1)

                wait_gather(b)
                transpose(b)
                fire_wb(c, b)

        for b in range(2):
            wait_wb(b)

    return k(idx3, table)


@jax.jit
def kernel(input_vector, table):
    tail = table[E_FULL:].reshape(-1)
    tlin = _relayout(table.T, tail).reshape(NUM_EMB, D)
    idx3 = input_vector.T.reshape(W, SB, 128)
    out5 = _gather(tlin, idx3)
    return out5.transpose(2, 4, 0, 1, 3).reshape(S, W, D)
